# baseline (speedup 1.0000x reference)
import jax
import jax.numpy as jnp
from jax.experimental import pallas as pl
from jax.experimental.pallas import tpu as pltpu

_E = 64
_D = 48
_C = 768
_ED = _E * _D
_NJ = 4
_JW = _ED // _NJ


def _moe_dense_kernel(x_ref, wr_ref, w1_ref, w2_ref, o_ref):
    j = pl.program_id(0)
    xb = x_ref[...].astype(jnp.bfloat16)
    logits = jax.lax.dot_general(
        xb, wr_ref[...].astype(jnp.bfloat16), (((1,), (1,)), ((), ())),
        preferred_element_type=jnp.float32)
    m = jnp.max(logits, axis=-1, keepdims=True)
    lane = jax.lax.broadcasted_iota(jnp.int32, logits.shape, 1)
    eid = jnp.min(jnp.where(logits == m, lane, _E), axis=-1)
    lo = (eid * _D)[:, None]

    h = jax.lax.dot_general(
        xb, w1_ref[...], (((1,), (0,)), ((), ())),
        preferred_element_type=jnp.float32)
    h = jnp.maximum(h, 0.0)
    h = h * h
    col = jax.lax.broadcasted_iota(jnp.int32, h.shape, 1) + j * _JW
    g = jnp.where((col >= lo) & (col < lo + _D), h, 0.0).astype(jnp.bfloat16)

    part = jax.lax.dot_general(
        g, w2_ref[...].astype(jnp.bfloat16), (((1,), (0,)), ((), ())),
        preferred_element_type=jnp.float32)

    @pl.when(j == 0)
    def _():
        o_ref[...] = part

    @pl.when(j > 0)
    def _():
        o_ref[...] += part


def kernel(x, Wr, W1, W2):
    B, T, C = x.shape
    N = B * T
    E, _, D = W1.shape
    x_flat = x.reshape(N, C)
    w1cat = W1.astype(jnp.bfloat16).transpose(1, 0, 2).reshape(C, E * D)
    w2cat = W2.reshape(E * D, C)

    out = pl.pallas_call(
        _moe_dense_kernel,
        grid=(_NJ,),
        in_specs=[
            pl.BlockSpec((N, C), lambda j: (0, 0)),
            pl.BlockSpec((E, C), lambda j: (0, 0)),
            pl.BlockSpec((C, _JW), lambda j: (0, j)),
            pl.BlockSpec((_JW, C), lambda j: (j, 0)),
        ],
        out_specs=pl.BlockSpec((N, C), lambda j: (0, 0)),
        out_shape=jax.ShapeDtypeStruct((N, C), jnp.float32),
        compiler_params=pltpu.CompilerParams(
            dimension_semantics=("arbitrary",)),
    )(x_flat, Wr, w1cat, w2cat)
    return out.reshape(B, T, C)


# speedup vs baseline: 1.1461x; 1.1461x over previous
import jax
import jax.numpy as jnp
from jax.experimental import pallas as pl
from jax.experimental.pallas import tpu as pltpu

_E = 64
_D = 48
_TM = 512


def _moe_dense_kernel(x_ref, wr_ref, w1_ref, w2_ref, o_ref):
    xb = x_ref[...].astype(jnp.bfloat16)
    logits = jax.lax.dot_general(
        xb, wr_ref[...].astype(jnp.bfloat16), (((1,), (1,)), ((), ())),
        preferred_element_type=jnp.float32)
    m = jnp.max(logits, axis=-1, keepdims=True)
    lane = jax.lax.broadcasted_iota(jnp.int32, logits.shape, 1)
    eid = jnp.min(jnp.where(logits == m, lane, _E), axis=-1)

    h = jax.lax.dot_general(
        xb, w1_ref[...], (((1,), (0,)), ((), ())),
        preferred_element_type=jnp.float32)
    h = jnp.maximum(h, 0.0)
    h = h * h
    col = jax.lax.broadcasted_iota(jnp.int32, h.shape, 1)
    rel = (col - (eid * _D)[:, None]).astype(jnp.uint32)
    g = jnp.where(rel < _D, h, 0.0).astype(jnp.bfloat16)

    o_ref[...] = jax.lax.dot_general(
        g, w2_ref[...].astype(jnp.bfloat16), (((1,), (0,)), ((), ())),
        preferred_element_type=jnp.float32)


def kernel(x, Wr, W1, W2):
    B, T, C = x.shape
    N = B * T
    E, _, D = W1.shape
    x_flat = x.reshape(N, C)
    w1cat = W1.astype(jnp.bfloat16).transpose(1, 0, 2).reshape(C, E * D)
    w2cat = W2.reshape(E * D, C)

    out = pl.pallas_call(
        _moe_dense_kernel,
        grid=(N // _TM,),
        in_specs=[
            pl.BlockSpec((_TM, C), lambda i: (i, 0)),
            pl.BlockSpec((E, C), lambda i: (0, 0)),
            pl.BlockSpec((C, E * D), lambda i: (0, 0)),
            pl.BlockSpec((E * D, C), lambda i: (0, 0)),
        ],
        out_specs=pl.BlockSpec((_TM, C), lambda i: (i, 0)),
        out_shape=jax.ShapeDtypeStruct((N, C), jnp.float32),
        compiler_params=pltpu.CompilerParams(
            dimension_semantics=("parallel",)),
    )(x_flat, Wr, w1cat, w2cat)
    return out.reshape(B, T, C)
